# Initial kernel scaffold; baseline (speedup 1.0000x reference)
import jax
import jax.numpy as jnp
from jax.experimental import pallas as pl


def kernel(xyz, patch_W1, patch_b1, patch_W2, patch_b2, patch_W3, patch_b3, pe, ln1_g, ln1_b, Wqkv, bqkv, Wo, bo, ln2_g, ln2_b, ffn_W1, ffn_b1, ffn_W2, ffn_b2, proj_W, proj_b, head_W1, head_b1, head_W2, head_b2):
    raise NotImplementedError("write your pallas kernel here")


# speedup vs baseline: 4.8964x; 4.8964x over previous
import functools

import jax
import jax.numpy as jnp
import numpy as np
from jax import lax
from jax.experimental import pallas as pl
from jax.experimental.pallas import tpu as pltpu
from jax.experimental.pallas import tpu_sc as plsc

B, N = 4, 32768
MP, KNN = 64, 128
EMB, DFF, NHEAD, DEPTH, NC = 256, 512, 8, 6, 10
DH = EMB // NHEAD
NW = 32
ROWS_PER_W = (B * MP) // NW

_PREC = lax.Precision.HIGHEST
_PREC_DIST = lax.Precision.DEFAULT
_F32 = jnp.float32


def _dot(a, b, prec=_PREC):
    return lax.dot_general(a, b, (((a.ndim - 1,), (0,)), ((), ())),
                           preferred_element_type=_F32, precision=prec)


def _k1_body(xyzT_ref, cen_ref, d_ref, v_ref, i_ref):
    Xt = xyzT_ref[0]
    C = cen_ref[0]
    xx = jnp.sum(Xt * Xt, axis=0, keepdims=True)
    cc = jnp.sum(C * C, axis=1, keepdims=True)
    P = lax.dot_general(C, Xt, (((1,), (0,)), ((), ())),
                        preferred_element_type=_F32, precision=_PREC_DIST)
    sq = (xx + cc) - 2.0 * P
    d = jnp.sqrt(jnp.maximum(sq, 0.0))
    d_ref[0] = d

    dbits = lax.bitcast_convert_type(d, jnp.int32)

    def vstep(i, V):
        Vc = V | lax.shift_left(1, 30 - i)
        cnt = jnp.sum(jnp.where(dbits < Vc, 1.0, 0.0), axis=1, keepdims=True)
        return jnp.where(cnt <= 127.0, Vc, V)

    V = lax.fori_loop(0, 31, vstep, jnp.zeros((MP, 1), jnp.int32))
    cnt_less = jnp.sum(jnp.where(dbits < V, 1.0, 0.0), axis=1, keepdims=True)
    need_eq = 128.0 - cnt_less
    ii = lax.broadcasted_iota(jnp.int32, (1, N), 1)

    def istep(i, I):
        Ic = I | lax.shift_left(1, 14 - i)
        cnt = jnp.sum(jnp.where((dbits == V) & (ii < Ic), 1.0, 0.0),
                      axis=1, keepdims=True)
        return jnp.where(cnt <= need_eq - 1.0, Ic, I)

    I = lax.fori_loop(0, 15, istep, jnp.zeros((MP, 1), jnp.int32))
    v_ref[0] = lax.bitcast_convert_type(V, _F32)
    i_ref[0] = I


def _run_k1(xyzT, centers):
    return pl.pallas_call(
        _k1_body,
        grid=(B,),
        in_specs=[
            pl.BlockSpec((1, 3, N), lambda b: (b, 0, 0)),
            pl.BlockSpec((1, MP, 3), lambda b: (b, 0, 0)),
        ],
        out_specs=[
            pl.BlockSpec((1, MP, N), lambda b: (b, 0, 0)),
            pl.BlockSpec((1, MP, 1), lambda b: (b, 0, 0)),
            pl.BlockSpec((1, MP, 1), lambda b: (b, 0, 0)),
        ],
        out_shape=[
            jax.ShapeDtypeStruct((B, MP, N), _F32),
            jax.ShapeDtypeStruct((B, MP, 1), _F32),
            jax.ShapeDtypeStruct((B, MP, 1), jnp.int32),
        ],
    )(xyzT, centers)


A_BLK = 4096


def _k1b_body(xyz_ref, w1_ref, b1_ref, a_ref):
    a_ref[0] = _dot(xyz_ref[0], w1_ref[...]) + b1_ref[...]


def _run_k1b(xyz, w1p, b1p):
    return pl.pallas_call(
        _k1b_body,
        grid=(B, N // A_BLK),
        in_specs=[
            pl.BlockSpec((1, A_BLK, 3), lambda b, n: (b, n, 0)),
            pl.BlockSpec((3, 128), lambda b, n: (0, 0)),
            pl.BlockSpec((128,), lambda b, n: (0,)),
        ],
        out_specs=[pl.BlockSpec((1, A_BLK, 128), lambda b, n: (b, n, 0))],
        out_shape=[jax.ShapeDtypeStruct((B, N, 128), _F32)],
    )(xyz, w1p, b1p)[0]


_GDN = lax.GatherDimensionNumbers(offset_dims=(), collapsed_slice_dims=(0,),
                                  start_index_map=(0,))


def _gather16(vec, idx):
    return lax.gather(vec, idx.reshape(16, 1), _GDN, slice_sizes=(1,),
                      mode=lax.GatherScatterMode.PROMISE_IN_BOUNDS)


def _sc_body(d_hbm, v_hbm, i_hbm, a_hbm, out_hbm,
             drow, vvm, ivm, idxpad, idx128, rows, sem):
    cid = lax.axis_index("c")
    sid = lax.axis_index("s")
    wid = sid * 2 + cid
    pltpu.sync_copy(v_hbm, vvm)
    pltpu.sync_copy(i_hbm, ivm)
    lane16 = lax.iota(jnp.int32, 16)

    def row_body(j, carry):
        r = wid * ROWS_PER_W + j
        bb = r // MP
        pltpu.sync_copy(d_hbm.at[r], drow)
        cbase = pl.multiple_of((r // 16) * 16, 16)
        lane = jnp.full((16,), r % 16, jnp.int32)
        Vb = _gather16(vvm[pl.ds(cbase, 16)], lane)
        Ib = _gather16(ivm[pl.ds(cbase, 16)], lane)
        boff = bb * N

        def step(i, base):
            dv = drow[pl.ds(i * 16, 16)]
            gidx = lane16 + i * 16
            m = (dv < Vb) | ((dv == Vb) & (gidx <= Ib))
            mi = jnp.where(m, 1, 0)
            t = mi
            for k in (1, 2, 4, 8):
                t = t + _gather16(t, (lane16 + k) & 15)

            def append(b0):
                acc = b0
                sbase = i * 16 + boff
                for tt in range(16):
                    idxpad[acc] = sbase + tt
                    acc = acc + mi[tt]
                return acc

            return lax.cond(t[0] > 0, append, lambda b0: b0, base)

        lax.fori_loop(0, N // 16, step, 0)
        for c in range(8):
            v = jnp.zeros((16,), jnp.int32)
            for tt in range(16):
                v = jnp.where(lane16 == tt, idxpad[c * 16 + tt], v)
            idx128[pl.ds(c * 16, 16)] = v
        pltpu.async_copy(a_hbm.at[idx128], rows, sem).wait()
        dst = out_hbm.at[pl.ds(pl.multiple_of(r * KNN, KNN), KNN)]
        pltpu.sync_copy(rows, dst)
        return carry

    lax.fori_loop(0, ROWS_PER_W, row_body, 0)


def _sc_select_gather(d2d, vflat, iflat, aflat):
    mesh = plsc.VectorSubcoreMesh(core_axis_name="c", subcore_axis_name="s")
    fn = pl.kernel(
        _sc_body,
        mesh=mesh,
        out_type=jax.ShapeDtypeStruct((B * MP * KNN, 128), _F32),
        scratch_types=[
            pltpu.VMEM((N,), _F32),
            pltpu.VMEM((B * MP,), _F32),
            pltpu.VMEM((B * MP,), jnp.int32),
            pltpu.SMEM((KNN + 16,), jnp.int32),
            pltpu.VMEM((KNN,), jnp.int32),
            pltpu.VMEM((KNN, 128), _F32),
            pltpu.SemaphoreType.DMA,
        ],
    )
    return fn(d2d, vflat, iflat, aflat)


def _ln_rep(x, g, b):
    mu = jnp.mean(x, axis=-1, keepdims=True)
    var = jnp.mean((x - mu) ** 2, axis=-1, keepdims=True)
    return (x - mu) / jnp.sqrt(var + 1e-5) * g + b


def _k3_body(g_ref, cen_ref, w1_ref, w2_ref, b2_ref, w3_ref, b3_ref, pe_ref,
             ln1g_ref, ln1b_ref, wqkv_ref, bqkv_ref, wo_ref, bo_ref,
             ln2g_ref, ln2b_ref, fw1_ref, fb1_ref, fw2_ref, fb2_ref,
             pw_ref, pb_ref, hw1_ref, hb1_ref, hw2_ref, hb2_ref,
             tab_ref):
    G = g_ref[0][:, :64]
    C = cen_ref[0]
    cw1 = _dot(C, w1_ref[...])
    cw1r = jnp.broadcast_to(cw1.reshape(MP, 1, 64),
                            (MP, KNN, 64)).reshape(MP * KNN, 64)
    f1 = jnp.maximum(G - cw1r, 0.0)
    f2 = jnp.maximum(_dot(f1, w2_ref[...]) + b2_ref[...], 0.0)
    f3 = _dot(f2, w3_ref[...]) + b3_ref[...]
    tok = jnp.max(f3.reshape(MP, KNN, EMB), axis=1)
    tok = tok + pe_ref[0]

    scale = 1.0 / float(np.sqrt(DH))
    for l in range(DEPTH):
        h = _ln_rep(tok, ln1g_ref[l], ln1b_ref[l])
        qkv = _dot(h, wqkv_ref[l]) + bqkv_ref[l]
        q, k, v = qkv[:, :EMB], qkv[:, EMB:2 * EMB], qkv[:, 2 * EMB:]
        outs = []
        for hh in range(NHEAD):
            sl = slice(hh * DH, (hh + 1) * DH)
            qh, kh, vh = q[:, sl], k[:, sl], v[:, sl]
            att = lax.dot_general(qh, kh, (((1,), (1,)), ((), ())),
                                  preferred_element_type=_F32,
                                  precision=_PREC) * scale
            mx = jnp.max(att, axis=-1, keepdims=True)
            e = jnp.exp(att - mx)
            att = e / jnp.sum(e, axis=-1, keepdims=True)
            outs.append(_dot(att, vh))
        o = jnp.concatenate(outs, axis=1)
        tok = tok + _dot(o, wo_ref[l]) + bo_ref[l]
        h2 = _ln_rep(tok, ln2g_ref[l], ln2b_ref[l])
        tok = tok + (_dot(jnp.maximum(_dot(h2, fw1_ref[l]) + fb1_ref[l], 0.0),
                          fw2_ref[l]) + fb2_ref[l])

    feats = _dot(tok, pw_ref[...]) + pb_ref[...]
    t3 = jnp.maximum(_dot(feats, hw1_ref[...]) + hb1_ref[...], 0.0)
    tab_ref[0] = _dot(t3, hw2_ref[...]) + hb2_ref[...]


def _run_k3(G, centers, args):
    (patch_W1, patch_W2, patch_b2, patch_W3, patch_b3, pe,
     ln1_g, ln1_b, Wqkv, bqkv, Wo, bo, ln2_g, ln2_b,
     ffn_W1, ffn_b1, ffn_W2, ffn_b2,
     proj_W, proj_b, head_W1, head_b1, head_W2, head_b2) = args

    def full(a):
        nd = a.ndim
        return pl.BlockSpec(a.shape, lambda b, _n=nd: (0,) * _n)

    weights = (patch_W1, patch_W2, patch_b2, patch_W3, patch_b3, pe,
               ln1_g, ln1_b, Wqkv, bqkv, Wo, bo, ln2_g, ln2_b,
               ffn_W1, ffn_b1, ffn_W2, ffn_b2,
               proj_W, proj_b, head_W1, head_b1, head_W2, head_b2)
    return pl.pallas_call(
        _k3_body,
        grid=(B,),
        in_specs=[pl.BlockSpec((1, MP * KNN, 128), lambda b: (b, 0, 0)),
                  pl.BlockSpec((1, MP, 3), lambda b: (b, 0, 0))] +
                 [full(w) for w in weights],
        out_specs=[pl.BlockSpec((1, MP, NC), lambda b: (b, 0, 0))],
        out_shape=[jax.ShapeDtypeStruct((B, MP, NC), _F32)],
    )(G, centers, *weights)[0]


K4_BLK = 4096


def _k4_body(xyz_ref, cenT_ref, tab_ref, out_ref):
    X = xyz_ref[0]
    Ct = cenT_ref[0]
    xx = jnp.sum(X * X, axis=1, keepdims=True)
    cc = jnp.sum(Ct * Ct, axis=0, keepdims=True)
    P = lax.dot_general(X, Ct, (((1,), (0,)), ((), ())),
                        preferred_element_type=_F32, precision=_PREC_DIST)
    sq = (xx + cc) - 2.0 * P
    d = jnp.sqrt(jnp.maximum(sq, 0.0))
    mn = jnp.min(d, axis=1, keepdims=True)
    li = lax.broadcasted_iota(jnp.int32, (K4_BLK, MP), 1)
    sel = jnp.where(d == mn, li, MP)
    am = jnp.min(sel, axis=1, keepdims=True)
    oh = jnp.where(li == am, 1.0, 0.0)
    out_ref[0] = lax.dot_general(oh, tab_ref[0], (((1,), (0,)), ((), ())),
                                 preferred_element_type=_F32,
                                 precision=lax.Precision.HIGHEST)


def _run_k4(xyz, cenT, tab):
    return pl.pallas_call(
        _k4_body,
        grid=(B, N // K4_BLK),
        in_specs=[
            pl.BlockSpec((1, K4_BLK, 3), lambda b, n: (b, n, 0)),
            pl.BlockSpec((1, 3, MP), lambda b, n: (b, 0, 0)),
            pl.BlockSpec((1, MP, NC), lambda b, n: (b, 0, 0)),
        ],
        out_specs=[pl.BlockSpec((1, K4_BLK, NC), lambda b, n: (b, n, 0))],
        out_shape=[jax.ShapeDtypeStruct((B, N, NC), _F32)],
    )(xyz, cenT, tab)[0]


def kernel(xyz, patch_W1, patch_b1, patch_W2, patch_b2, patch_W3, patch_b3,
           pe, ln1_g, ln1_b, Wqkv, bqkv, Wo, bo, ln2_g, ln2_b,
           ffn_W1, ffn_b1, ffn_W2, ffn_b2, proj_W, proj_b,
           head_W1, head_b1, head_W2, head_b2):
    idx_c = jnp.linspace(0.0, N - 1, MP).astype(jnp.int32)
    centers = jnp.take(xyz, idx_c, axis=1)
    xyzT = jnp.swapaxes(xyz, 1, 2)
    cenT = jnp.swapaxes(centers, 1, 2)
    w1p = jnp.pad(patch_W1, ((0, 0), (0, 64)))
    b1p = jnp.pad(patch_b1, (0, 64))

    d, V, I = _run_k1(xyzT, centers)
    A = _run_k1b(xyz, w1p, b1p)
    G = _sc_select_gather(d.reshape(B * MP, N),
                          V.reshape(B * MP),
                          I.reshape(B * MP),
                          A.reshape(B * N, 128))
    tab = _run_k3(G.reshape(B, MP * KNN, 128), centers,
                  (patch_W1, patch_W2, patch_b2, patch_W3, patch_b3, pe,
                   ln1_g, ln1_b, Wqkv, bqkv, Wo, bo, ln2_g, ln2_b,
                   ffn_W1, ffn_b1, ffn_W2, ffn_b2,
                   proj_W, proj_b, head_W1, head_b1, head_W2, head_b2))
    return _run_k4(xyz, cenT, tab)


# speedup vs baseline: 5.7919x; 1.1829x over previous
import functools

import jax
import jax.numpy as jnp
import numpy as np
from jax import lax
from jax.experimental import pallas as pl
from jax.experimental.pallas import tpu as pltpu
from jax.experimental.pallas import tpu_sc as plsc

B, N = 4, 32768
MP, KNN = 64, 128
EMB, DFF, NHEAD, DEPTH, NC = 256, 512, 8, 6, 10
DH = EMB // NHEAD
NW = 32
ROWS_PER_W = (B * MP) // NW

_PREC = lax.Precision.HIGHEST
_PREC_DIST = lax.Precision.DEFAULT
_F32 = jnp.float32


def _dot(a, b, prec=_PREC):
    return lax.dot_general(a, b, (((a.ndim - 1,), (0,)), ((), ())),
                           preferred_element_type=_F32, precision=prec)


def _k1_body(xyzT_ref, cen_ref, d_ref, v_ref, i_ref):
    Xt = xyzT_ref[0]
    C = cen_ref[0]
    xx = jnp.sum(Xt * Xt, axis=0, keepdims=True)
    cc = jnp.sum(C * C, axis=1, keepdims=True)
    P = lax.dot_general(C, Xt, (((1,), (0,)), ((), ())),
                        preferred_element_type=_F32, precision=_PREC_DIST)
    sq = (xx + cc) - 2.0 * P
    d = jnp.sqrt(jnp.maximum(sq, 0.0))
    d_ref[0] = d

    dbits = lax.bitcast_convert_type(d, jnp.int32)

    def vstep(i, V):
        Vc = V | lax.shift_left(1, 30 - i)
        cnt = jnp.sum(jnp.where(dbits < Vc, 1.0, 0.0), axis=1, keepdims=True)
        return jnp.where(cnt <= 127.0, Vc, V)

    V = lax.fori_loop(0, 31, vstep, jnp.zeros((MP, 1), jnp.int32))
    cnt_less = jnp.sum(jnp.where(dbits < V, 1.0, 0.0), axis=1, keepdims=True)
    need_eq = 128.0 - cnt_less
    ii = lax.broadcasted_iota(jnp.int32, (1, N), 1)

    def istep(i, I):
        Ic = I | lax.shift_left(1, 14 - i)
        cnt = jnp.sum(jnp.where((dbits == V) & (ii < Ic), 1.0, 0.0),
                      axis=1, keepdims=True)
        return jnp.where(cnt <= need_eq - 1.0, Ic, I)

    I = lax.fori_loop(0, 15, istep, jnp.zeros((MP, 1), jnp.int32))
    v_ref[0] = lax.bitcast_convert_type(V, _F32)
    i_ref[0] = I


def _run_k1(xyzT, centers):
    return pl.pallas_call(
        _k1_body,
        grid=(B,),
        in_specs=[
            pl.BlockSpec((1, 3, N), lambda b: (b, 0, 0)),
            pl.BlockSpec((1, MP, 3), lambda b: (b, 0, 0)),
        ],
        out_specs=[
            pl.BlockSpec((1, MP, N), lambda b: (b, 0, 0)),
            pl.BlockSpec((1, MP, 1), lambda b: (b, 0, 0)),
            pl.BlockSpec((1, MP, 1), lambda b: (b, 0, 0)),
        ],
        out_shape=[
            jax.ShapeDtypeStruct((B, MP, N), _F32),
            jax.ShapeDtypeStruct((B, MP, 1), _F32),
            jax.ShapeDtypeStruct((B, MP, 1), jnp.int32),
        ],
    )(xyzT, centers)


_GDN = lax.GatherDimensionNumbers(offset_dims=(), collapsed_slice_dims=(0,),
                                  start_index_map=(0,))


def _gather16(vec, idx):
    return lax.gather(vec, idx.reshape(16, 1), _GDN, slice_sizes=(1,),
                      mode=lax.GatherScatterMode.PROMISE_IN_BOUNDS)


def _sc_body(d_hbm, v_hbm, i_hbm, a_hbm, out_hbm,
             drow, vvm, ivm, idxpad, idx128, rows, sem):
    cid = lax.axis_index("c")
    sid = lax.axis_index("s")
    wid = sid * 2 + cid
    pltpu.sync_copy(v_hbm, vvm)
    pltpu.sync_copy(i_hbm, ivm)
    lane16 = lax.iota(jnp.int32, 16)

    def row_body(j, carry):
        r = wid * ROWS_PER_W + j
        bb = r // MP
        pltpu.sync_copy(d_hbm.at[r], drow)
        cbase = pl.multiple_of((r // 16) * 16, 16)
        lane = jnp.full((16,), r % 16, jnp.int32)
        Vb = _gather16(vvm[pl.ds(cbase, 16)], lane)
        Ib = _gather16(ivm[pl.ds(cbase, 16)], lane)
        boff = bb * N

        def step(i, base):
            dv = drow[pl.ds(i * 16, 16)]
            gidx = lane16 + i * 16
            m = (dv < Vb) | ((dv == Vb) & (gidx <= Ib))
            mi = jnp.where(m, 1, 0)
            t = mi
            for k in (1, 2, 4, 8):
                t = t + _gather16(t, (lane16 + k) & 15)

            def append(b0):
                acc = b0
                sbase = i * 16 + boff
                for tt in range(16):
                    idxpad[acc] = sbase + tt
                    acc = acc + mi[tt]
                return acc

            return lax.cond(t[0] > 0, append, lambda b0: b0, base)

        lax.fori_loop(0, N // 16, step, 0)
        for c in range(8):
            v = jnp.zeros((16,), jnp.int32)
            for tt in range(16):
                v = jnp.where(lane16 == tt, idxpad[c * 16 + tt], v)
            idx128[pl.ds(c * 16, 16)] = v
        pltpu.async_copy(a_hbm.at[idx128], rows, sem).wait()
        dst = out_hbm.at[pl.ds(pl.multiple_of(r * KNN, KNN), KNN)]
        pltpu.sync_copy(rows, dst)
        return carry

    lax.fori_loop(0, ROWS_PER_W, row_body, 0)


def _sc_select_gather(d2d, vflat, iflat, aflat):
    mesh = plsc.VectorSubcoreMesh(core_axis_name="c", subcore_axis_name="s")
    fn = pl.kernel(
        _sc_body,
        mesh=mesh,
        out_type=jax.ShapeDtypeStruct((B * MP * KNN, 128), _F32),
        scratch_types=[
            pltpu.VMEM((N,), _F32),
            pltpu.VMEM((B * MP,), _F32),
            pltpu.VMEM((B * MP,), jnp.int32),
            pltpu.SMEM((KNN + 16,), jnp.int32),
            pltpu.VMEM((KNN,), jnp.int32),
            pltpu.VMEM((KNN, 128), _F32),
            pltpu.SemaphoreType.DMA,
        ],
    )
    return fn(d2d, vflat, iflat, aflat)


def _ln_rep(x, g, b):
    mu = jnp.mean(x, axis=-1, keepdims=True)
    var = jnp.mean((x - mu) ** 2, axis=-1, keepdims=True)
    return (x - mu) / jnp.sqrt(var + 1e-5) * g + b


def _k3_body(g_ref, cen_ref, w1_ref, b1_ref, w2_ref, b2_ref, w3_ref, b3_ref,
             pe_ref,
             ln1g_ref, ln1b_ref, wqkv_ref, bqkv_ref, wo_ref, bo_ref,
             ln2g_ref, ln2b_ref, fw1_ref, fb1_ref, fw2_ref, fb2_ref,
             pw_ref, pb_ref, hw1_ref, hb1_ref, hw2_ref, hb2_ref,
             tab_ref):
    Gx = g_ref[0][:, :3]
    C = cen_ref[0]
    crep = jnp.broadcast_to(C.reshape(MP, 1, 3),
                            (MP, KNN, 3)).reshape(MP * KNN, 3)
    local = Gx - crep
    f1 = jnp.maximum(_dot(local, w1_ref[...], _PREC_DIST) + b1_ref[...], 0.0)
    f2 = jnp.maximum(_dot(f1, w2_ref[...], _PREC_DIST) + b2_ref[...], 0.0)
    f3 = _dot(f2, w3_ref[...], _PREC_DIST) + b3_ref[...]
    tok = jnp.max(f3.reshape(MP, KNN, EMB), axis=1)
    tok = tok + pe_ref[0]

    scale = 1.0 / float(np.sqrt(DH))
    for l in range(DEPTH):
        h = _ln_rep(tok, ln1g_ref[l], ln1b_ref[l])
        qkv = _dot(h, wqkv_ref[l], _PREC_DIST) + bqkv_ref[l]
        q, k, v = qkv[:, :EMB], qkv[:, EMB:2 * EMB], qkv[:, 2 * EMB:]
        outs = []
        for hh in range(NHEAD):
            sl = slice(hh * DH, (hh + 1) * DH)
            qh, kh, vh = q[:, sl], k[:, sl], v[:, sl]
            att = lax.dot_general(qh, kh, (((1,), (1,)), ((), ())),
                                  preferred_element_type=_F32,
                                  precision=_PREC_DIST) * scale
            mx = jnp.max(att, axis=-1, keepdims=True)
            e = jnp.exp(att - mx)
            att = e / jnp.sum(e, axis=-1, keepdims=True)
            outs.append(_dot(att, vh, _PREC_DIST))
        o = jnp.concatenate(outs, axis=1)
        tok = tok + _dot(o, wo_ref[l], _PREC_DIST) + bo_ref[l]
        h2 = _ln_rep(tok, ln2g_ref[l], ln2b_ref[l])
        tok = tok + (_dot(jnp.maximum(_dot(h2, fw1_ref[l], _PREC_DIST)
                                      + fb1_ref[l], 0.0),
                          fw2_ref[l], _PREC_DIST) + fb2_ref[l])

    feats = _dot(tok, pw_ref[...], _PREC_DIST) + pb_ref[...]
    t3 = jnp.maximum(_dot(feats, hw1_ref[...], _PREC_DIST) + hb1_ref[...], 0.0)
    tab_ref[0] = _dot(t3, hw2_ref[...], _PREC_DIST) + hb2_ref[...]


def _run_k3(G, centers, args):
    def full(a):
        nd = a.ndim
        return pl.BlockSpec(a.shape, lambda b, _n=nd: (0,) * _n)

    weights = args
    return pl.pallas_call(
        _k3_body,
        grid=(B,),
        in_specs=[pl.BlockSpec((1, MP * KNN, 128), lambda b: (b, 0, 0)),
                  pl.BlockSpec((1, MP, 3), lambda b: (b, 0, 0))] +
                 [full(w) for w in weights],
        out_specs=[pl.BlockSpec((1, MP, NC), lambda b: (b, 0, 0))],
        out_shape=[jax.ShapeDtypeStruct((B, MP, NC), _F32)],
    )(G, centers, *weights)[0]


K4_BLK = 4096


def _k4_body(xyz_ref, cenT_ref, tab_ref, out_ref):
    X = xyz_ref[0]
    Ct = cenT_ref[0]
    xx = jnp.sum(X * X, axis=1, keepdims=True)
    cc = jnp.sum(Ct * Ct, axis=0, keepdims=True)
    P = lax.dot_general(X, Ct, (((1,), (0,)), ((), ())),
                        preferred_element_type=_F32, precision=_PREC_DIST)
    sq = (xx + cc) - 2.0 * P
    d = jnp.sqrt(jnp.maximum(sq, 0.0))
    mn = jnp.min(d, axis=1, keepdims=True)
    li = lax.broadcasted_iota(jnp.int32, (K4_BLK, MP), 1)
    sel = jnp.where(d == mn, li, MP)
    am = jnp.min(sel, axis=1, keepdims=True)
    oh = jnp.where(li == am, 1.0, 0.0)
    out_ref[0] = lax.dot_general(oh, tab_ref[0], (((1,), (0,)), ((), ())),
                                 preferred_element_type=_F32,
                                 precision=lax.Precision.HIGHEST)


def _run_k4(xyz, cenT, tab):
    return pl.pallas_call(
        _k4_body,
        grid=(B, N // K4_BLK),
        in_specs=[
            pl.BlockSpec((1, K4_BLK, 3), lambda b, n: (b, n, 0)),
            pl.BlockSpec((1, 3, MP), lambda b, n: (b, 0, 0)),
            pl.BlockSpec((1, MP, NC), lambda b, n: (b, 0, 0)),
        ],
        out_specs=[pl.BlockSpec((1, K4_BLK, NC), lambda b, n: (b, n, 0))],
        out_shape=[jax.ShapeDtypeStruct((B, N, NC), _F32)],
    )(xyz, cenT, tab)[0]


def kernel(xyz, patch_W1, patch_b1, patch_W2, patch_b2, patch_W3, patch_b3,
           pe, ln1_g, ln1_b, Wqkv, bqkv, Wo, bo, ln2_g, ln2_b,
           ffn_W1, ffn_b1, ffn_W2, ffn_b2, proj_W, proj_b,
           head_W1, head_b1, head_W2, head_b2):
    idx_c = jnp.linspace(0.0, N - 1, MP).astype(jnp.int32)
    centers = jnp.take(xyz, idx_c, axis=1)
    xyzT = jnp.swapaxes(xyz, 1, 2)
    cenT = jnp.swapaxes(centers, 1, 2)
    xyzpad = jnp.pad(xyz, ((0, 0), (0, 0), (0, 125))).reshape(B * N, 128)

    d, V, I = _run_k1(xyzT, centers)
    G = _sc_select_gather(d.reshape(B * MP, N),
                          V.reshape(B * MP),
                          I.reshape(B * MP),
                          xyzpad)
    tab = _run_k3(G.reshape(B, MP * KNN, 128), centers,
                  (patch_W1, patch_b1, patch_W2, patch_b2, patch_W3, patch_b3,
                   pe, ln1_g, ln1_b, Wqkv, bqkv, Wo, bo, ln2_g, ln2_b,
                   ffn_W1, ffn_b1, ffn_W2, ffn_b2,
                   proj_W, proj_b, head_W1, head_b1, head_W2, head_b2))
    return _run_k4(xyz, cenT, tab)


# speedup vs baseline: 6.2793x; 1.0842x over previous
import functools

import jax
import jax.numpy as jnp
import numpy as np
from jax import lax
from jax.experimental import pallas as pl
from jax.experimental.pallas import tpu as pltpu
from jax.experimental.pallas import tpu_sc as plsc

B, N = 4, 32768
MP, KNN = 64, 128
EMB, DFF, NHEAD, DEPTH, NC = 256, 512, 8, 6, 10
DH = EMB // NHEAD
NW = 32
ROWS_PER_W = (B * MP) // NW

_PREC = lax.Precision.HIGHEST
_PREC_DIST = lax.Precision.DEFAULT
_F32 = jnp.float32


def _dot(a, b, prec=_PREC):
    return lax.dot_general(a, b, (((a.ndim - 1,), (0,)), ((), ())),
                           preferred_element_type=_F32, precision=prec)


MP_BLK = 16


def _k1_body(xyzT_ref, cen_ref, w_ref):
    Xt = xyzT_ref[0]
    C = cen_ref[0]
    xx = jnp.sum(Xt * Xt, axis=0, keepdims=True)
    cc = jnp.sum(C * C, axis=1, keepdims=True)
    P = lax.dot_general(C, Xt, (((1,), (0,)), ((), ())),
                        preferred_element_type=_F32, precision=_PREC_DIST)
    sq = (xx + cc) - 2.0 * P
    d = jnp.sqrt(jnp.maximum(sq, 0.0))

    dbits = lax.bitcast_convert_type(d, jnp.int32)

    def vstep(i, V):
        Vc = V | lax.shift_left(1, 30 - i)
        cnt = jnp.sum(jnp.where(dbits < Vc, 1.0, 0.0), axis=1, keepdims=True)
        return jnp.where(cnt <= 127.0, Vc, V)

    V = lax.fori_loop(0, 31, vstep, jnp.zeros((MP_BLK, 1), jnp.int32))
    cnt_less = jnp.sum(jnp.where(dbits < V, 1.0, 0.0), axis=1, keepdims=True)
    need_eq = 128.0 - cnt_less
    ii = lax.broadcasted_iota(jnp.int32, (1, N), 1)

    def istep(i, I):
        Ic = I | lax.shift_left(1, 14 - i)
        cnt = jnp.sum(jnp.where((dbits == V) & (ii < Ic), 1.0, 0.0),
                      axis=1, keepdims=True)
        return jnp.where(cnt <= need_eq - 1.0, Ic, I)

    I = lax.fori_loop(0, 15, istep, jnp.zeros((MP_BLK, 1), jnp.int32))

    mask = (dbits < V) | ((dbits == V) & (ii <= I))
    pw = lax.shift_left(1, ii & 15)
    wf = jnp.where(mask, pw, 0).astype(_F32)
    words = jnp.sum(wf.reshape(MP_BLK, N // 16, 16), axis=2)
    w_ref[0] = words.astype(jnp.int32)


def _run_k1(xyzT, centers):
    return pl.pallas_call(
        _k1_body,
        grid=(B, MP // MP_BLK),
        in_specs=[
            pl.BlockSpec((1, 3, N), lambda b, m: (b, 0, 0)),
            pl.BlockSpec((1, MP_BLK, 3), lambda b, m: (b, m, 0)),
        ],
        out_specs=[
            pl.BlockSpec((1, MP_BLK, N // 16), lambda b, m: (b, m, 0)),
        ],
        out_shape=[
            jax.ShapeDtypeStruct((B, MP, N // 16), jnp.int32),
        ],
    )(xyzT, centers)[0]


_GDN = lax.GatherDimensionNumbers(offset_dims=(), collapsed_slice_dims=(0,),
                                  start_index_map=(0,))


def _gather16(vec, idx):
    return lax.gather(vec, idx.reshape(16, 1), _GDN, slice_sizes=(1,),
                      mode=lax.GatherScatterMode.PROMISE_IN_BOUNDS)


NWORDS = N // 16


def _sc_body(w_hbm, a_hbm, out_hbm, mrow, idxpad, idx128, rows, sem):
    cid = lax.axis_index("c")
    sid = lax.axis_index("s")
    wid = sid * 2 + cid
    lane16 = lax.iota(jnp.int32, 16)

    def row_body(j, carry):
        r = wid * ROWS_PER_W + j
        bb = r // MP
        pltpu.sync_copy(w_hbm.at[r], mrow)
        boff = bb * N

        def step(s, acc):
            wv = mrow[pl.ds(s * 16, 16)]
            t = wv
            for k in (1, 2, 4, 8):
                t = t | _gather16(t, (lane16 + k) & 15)

            def slow(a0):
                a = a0
                for tt in range(16):
                    w0 = wv[tt]
                    ebase = (s * 16 + tt) * 16 + boff
                    for bit in range(16):
                        idxpad[a] = ebase + bit
                        a = a + (lax.shift_right_logical(w0, bit) & 1)
                return a

            return lax.cond(t[0] != 0, slow, lambda a0: a0, acc)

        lax.fori_loop(0, NWORDS // 16, step, 0)
        for c in range(8):
            v = jnp.zeros((16,), jnp.int32)
            for tt in range(16):
                v = jnp.where(lane16 == tt, idxpad[c * 16 + tt], v)
            idx128[pl.ds(c * 16, 16)] = v
        pltpu.async_copy(a_hbm.at[idx128], rows, sem).wait()
        dst = out_hbm.at[pl.ds(pl.multiple_of(r * KNN, KNN), KNN)]
        pltpu.sync_copy(rows, dst)
        return carry

    lax.fori_loop(0, ROWS_PER_W, row_body, 0)


def _sc_select_gather(words2d, aflat):
    mesh = plsc.VectorSubcoreMesh(core_axis_name="c", subcore_axis_name="s")
    fn = pl.kernel(
        _sc_body,
        mesh=mesh,
        out_type=jax.ShapeDtypeStruct((B * MP * KNN, 128), _F32),
        scratch_types=[
            pltpu.VMEM((NWORDS,), jnp.int32),
            pltpu.SMEM((KNN + 1,), jnp.int32),
            pltpu.VMEM((KNN,), jnp.int32),
            pltpu.VMEM((KNN, 128), _F32),
            pltpu.SemaphoreType.DMA,
        ],
    )
    return fn(words2d, aflat)


def _ln_rep(x, g, b):
    mu = jnp.mean(x, axis=-1, keepdims=True)
    var = jnp.mean((x - mu) ** 2, axis=-1, keepdims=True)
    return (x - mu) / jnp.sqrt(var + 1e-5) * g + b


def _k3_body(g_ref, cen_ref, w1_ref, b1_ref, w2_ref, b2_ref, w3_ref, b3_ref,
             pe_ref,
             ln1g_ref, ln1b_ref, wqkv_ref, bqkv_ref, wo_ref, bo_ref,
             ln2g_ref, ln2b_ref, fw1_ref, fb1_ref, fw2_ref, fb2_ref,
             pw_ref, pb_ref, hw1_ref, hb1_ref, hw2_ref, hb2_ref,
             tab_ref):
    Gx = g_ref[0][:, :3]
    C = cen_ref[0]
    crep = jnp.broadcast_to(C.reshape(MP, 1, 3),
                            (MP, KNN, 3)).reshape(MP * KNN, 3)
    local = Gx - crep
    f1 = jnp.maximum(_dot(local, w1_ref[...], _PREC_DIST) + b1_ref[...], 0.0)
    f2 = jnp.maximum(_dot(f1, w2_ref[...], _PREC_DIST) + b2_ref[...], 0.0)
    f3 = _dot(f2, w3_ref[...], _PREC_DIST) + b3_ref[...]
    tok = jnp.max(f3.reshape(MP, KNN, EMB), axis=1)
    tok = tok + pe_ref[0]

    scale = 1.0 / float(np.sqrt(DH))
    for l in range(DEPTH):
        h = _ln_rep(tok, ln1g_ref[l], ln1b_ref[l])
        qkv = _dot(h, wqkv_ref[l], _PREC_DIST) + bqkv_ref[l]
        q, k, v = qkv[:, :EMB], qkv[:, EMB:2 * EMB], qkv[:, 2 * EMB:]
        outs = []
        for hh in range(NHEAD):
            sl = slice(hh * DH, (hh + 1) * DH)
            qh, kh, vh = q[:, sl], k[:, sl], v[:, sl]
            att = lax.dot_general(qh, kh, (((1,), (1,)), ((), ())),
                                  preferred_element_type=_F32,
                                  precision=_PREC_DIST) * scale
            mx = jnp.max(att, axis=-1, keepdims=True)
            e = jnp.exp(att - mx)
            att = e / jnp.sum(e, axis=-1, keepdims=True)
            outs.append(_dot(att, vh, _PREC_DIST))
        o = jnp.concatenate(outs, axis=1)
        tok = tok + _dot(o, wo_ref[l], _PREC_DIST) + bo_ref[l]
        h2 = _ln_rep(tok, ln2g_ref[l], ln2b_ref[l])
        tok = tok + (_dot(jnp.maximum(_dot(h2, fw1_ref[l], _PREC_DIST)
                                      + fb1_ref[l], 0.0),
                          fw2_ref[l], _PREC_DIST) + fb2_ref[l])

    feats = _dot(tok, pw_ref[...], _PREC_DIST) + pb_ref[...]
    t3 = jnp.maximum(_dot(feats, hw1_ref[...], _PREC_DIST) + hb1_ref[...], 0.0)
    tab_ref[0] = _dot(t3, hw2_ref[...], _PREC_DIST) + hb2_ref[...]


def _run_k3(G, centers, args):
    def full(a):
        nd = a.ndim
        return pl.BlockSpec(a.shape, lambda b, _n=nd: (0,) * _n)

    weights = args
    return pl.pallas_call(
        _k3_body,
        grid=(B,),
        in_specs=[pl.BlockSpec((1, MP * KNN, 128), lambda b: (b, 0, 0)),
                  pl.BlockSpec((1, MP, 3), lambda b: (b, 0, 0))] +
                 [full(w) for w in weights],
        out_specs=[pl.BlockSpec((1, MP, NC), lambda b: (b, 0, 0))],
        out_shape=[jax.ShapeDtypeStruct((B, MP, NC), _F32)],
    )(G, centers, *weights)[0]


K4_BLK = 4096


def _k4_body(xyz_ref, cenT_ref, tab_ref, out_ref):
    X = xyz_ref[0]
    Ct = cenT_ref[0]
    xx = jnp.sum(X * X, axis=1, keepdims=True)
    cc = jnp.sum(Ct * Ct, axis=0, keepdims=True)
    P = lax.dot_general(X, Ct, (((1,), (0,)), ((), ())),
                        preferred_element_type=_F32, precision=_PREC_DIST)
    sq = (xx + cc) - 2.0 * P
    d = jnp.sqrt(jnp.maximum(sq, 0.0))
    mn = jnp.min(d, axis=1, keepdims=True)
    li = lax.broadcasted_iota(jnp.int32, (K4_BLK, MP), 1)
    sel = jnp.where(d == mn, li, MP)
    am = jnp.min(sel, axis=1, keepdims=True)
    oh = jnp.where(li == am, 1.0, 0.0)
    out_ref[0] = lax.dot_general(oh, tab_ref[0], (((1,), (0,)), ((), ())),
                                 preferred_element_type=_F32,
                                 precision=lax.Precision.HIGHEST)


def _run_k4(xyz, cenT, tab):
    return pl.pallas_call(
        _k4_body,
        grid=(B, N // K4_BLK),
        in_specs=[
            pl.BlockSpec((1, K4_BLK, 3), lambda b, n: (b, n, 0)),
            pl.BlockSpec((1, 3, MP), lambda b, n: (b, 0, 0)),
            pl.BlockSpec((1, MP, NC), lambda b, n: (b, 0, 0)),
        ],
        out_specs=[pl.BlockSpec((1, K4_BLK, NC), lambda b, n: (b, n, 0))],
        out_shape=[jax.ShapeDtypeStruct((B, N, NC), _F32)],
    )(xyz, cenT, tab)[0]


def kernel(xyz, patch_W1, patch_b1, patch_W2, patch_b2, patch_W3, patch_b3,
           pe, ln1_g, ln1_b, Wqkv, bqkv, Wo, bo, ln2_g, ln2_b,
           ffn_W1, ffn_b1, ffn_W2, ffn_b2, proj_W, proj_b,
           head_W1, head_b1, head_W2, head_b2):
    idx_c = jnp.linspace(0.0, N - 1, MP).astype(jnp.int32)
    centers = jnp.take(xyz, idx_c, axis=1)
    xyzT = jnp.swapaxes(xyz, 1, 2)
    cenT = jnp.swapaxes(centers, 1, 2)
    xyzpad = jnp.pad(xyz, ((0, 0), (0, 0), (0, 125))).reshape(B * N, 128)

    words = _run_k1(xyzT, centers)
    G = _sc_select_gather(words.reshape(B * MP, N // 16), xyzpad)
    tab = _run_k3(G.reshape(B, MP * KNN, 128), centers,
                  (patch_W1, patch_b1, patch_W2, patch_b2, patch_W3, patch_b3,
                   pe, ln1_g, ln1_b, Wqkv, bqkv, Wo, bo, ln2_g, ln2_b,
                   ffn_W1, ffn_b1, ffn_W2, ffn_b2,
                   proj_W, proj_b, head_W1, head_b1, head_W2, head_b2))
    return _run_k4(xyz, cenT, tab)


# speedup vs baseline: 6.3421x; 1.0100x over previous
import functools

import jax
import jax.numpy as jnp
import numpy as np
from jax import lax
from jax.experimental import pallas as pl
from jax.experimental.pallas import tpu as pltpu
from jax.experimental.pallas import tpu_sc as plsc

B, N = 4, 32768
MP, KNN = 64, 128
EMB, DFF, NHEAD, DEPTH, NC = 256, 512, 8, 6, 10
DH = EMB // NHEAD
NW = 32
ROWS_PER_W = (B * MP) // NW

_PREC = lax.Precision.HIGHEST
_PREC_DIST = lax.Precision.DEFAULT
_F32 = jnp.float32


def _dot(a, b, prec=_PREC):
    return lax.dot_general(a, b, (((a.ndim - 1,), (0,)), ((), ())),
                           preferred_element_type=_F32, precision=prec)


MP_BLK = 16


def _k1_body(xyzT_ref, cen_ref, w_ref):
    Xt = xyzT_ref[0]
    C = cen_ref[0]
    xx = jnp.sum(Xt * Xt, axis=0, keepdims=True)
    cc = jnp.sum(C * C, axis=1, keepdims=True)
    P = lax.dot_general(C, Xt, (((1,), (0,)), ((), ())),
                        preferred_element_type=_F32, precision=_PREC_DIST)
    sq = (xx + cc) - 2.0 * P
    dbits = lax.bitcast_convert_type(jnp.maximum(sq, 0.0), jnp.int32)

    def vstep(i, V):
        Vc = V | lax.shift_left(1, 30 - i)
        cnt = jnp.sum(jnp.where(dbits < Vc, 1.0, 0.0), axis=1, keepdims=True)
        return jnp.where(cnt <= 127.0, Vc, V)

    V = lax.fori_loop(0, 31, vstep, jnp.zeros((MP_BLK, 1), jnp.int32))
    cnt_less = jnp.sum(jnp.where(dbits < V, 1.0, 0.0), axis=1, keepdims=True)
    need_eq = 128.0 - cnt_less
    ii = lax.broadcasted_iota(jnp.int32, (1, N), 1)

    def istep(i, I):
        Ic = I | lax.shift_left(1, 14 - i)
        cnt = jnp.sum(jnp.where((dbits == V) & (ii < Ic), 1.0, 0.0),
                      axis=1, keepdims=True)
        return jnp.where(cnt <= need_eq - 1.0, Ic, I)

    I = lax.fori_loop(0, 15, istep, jnp.zeros((MP_BLK, 1), jnp.int32))

    mask = (dbits < V) | ((dbits == V) & (ii <= I))
    pw = lax.shift_left(1, ii & 15)
    wf = jnp.where(mask, pw, 0).astype(_F32)
    words = jnp.sum(wf.reshape(MP_BLK, N // 16, 16), axis=2)
    w_ref[0] = words.astype(jnp.int32)


def _run_k1(xyzT, centers):
    return pl.pallas_call(
        _k1_body,
        grid=(B, MP // MP_BLK),
        in_specs=[
            pl.BlockSpec((1, 3, N), lambda b, m: (b, 0, 0)),
            pl.BlockSpec((1, MP_BLK, 3), lambda b, m: (b, m, 0)),
        ],
        out_specs=[
            pl.BlockSpec((1, MP_BLK, N // 16), lambda b, m: (b, m, 0)),
        ],
        out_shape=[
            jax.ShapeDtypeStruct((B, MP, N // 16), jnp.int32),
        ],
    )(xyzT, centers)[0]


_GDN = lax.GatherDimensionNumbers(offset_dims=(), collapsed_slice_dims=(0,),
                                  start_index_map=(0,))


def _gather16(vec, idx):
    return lax.gather(vec, idx.reshape(16, 1), _GDN, slice_sizes=(1,),
                      mode=lax.GatherScatterMode.PROMISE_IN_BOUNDS)


NWORDS = N // 16


def _sc_body(w_hbm, a_hbm, out_hbm, mrow, idxpad, idx128, rows, sem):
    cid = lax.axis_index("c")
    sid = lax.axis_index("s")
    wid = sid * 2 + cid
    lane16 = lax.iota(jnp.int32, 16)

    def row_body(j, carry):
        r = wid * ROWS_PER_W + j
        bb = r // MP
        pltpu.sync_copy(w_hbm.at[r], mrow)
        boff = bb * N

        def step(s, acc):
            wv = mrow[pl.ds(s * 16, 16)]
            t = wv
            for k in (1, 2, 4, 8):
                t = t | _gather16(t, (lane16 + k) & 15)

            def slow(a0):
                a = a0
                for tt in range(16):
                    w0 = wv[tt]
                    ebase = (s * 16 + tt) * 16 + boff
                    for bit in range(16):
                        idxpad[a] = ebase + bit
                        a = a + (lax.shift_right_logical(w0, bit) & 1)
                return a

            return lax.cond(t[0] != 0, slow, lambda a0: a0, acc)

        lax.fori_loop(0, NWORDS // 16, step, 0)
        for c in range(8):
            v = jnp.zeros((16,), jnp.int32)
            for tt in range(16):
                v = jnp.where(lane16 == tt, idxpad[c * 16 + tt], v)
            idx128[pl.ds(c * 16, 16)] = v
        pltpu.async_copy(a_hbm.at[idx128], rows, sem).wait()
        dst = out_hbm.at[pl.ds(pl.multiple_of(r * KNN, KNN), KNN)]
        pltpu.sync_copy(rows, dst)
        return carry

    lax.fori_loop(0, ROWS_PER_W, row_body, 0)


def _sc_select_gather(words2d, aflat):
    mesh = plsc.VectorSubcoreMesh(core_axis_name="c", subcore_axis_name="s")
    fn = pl.kernel(
        _sc_body,
        mesh=mesh,
        out_type=jax.ShapeDtypeStruct((B * MP * KNN, 128), _F32),
        scratch_types=[
            pltpu.VMEM((NWORDS,), jnp.int32),
            pltpu.SMEM((KNN + 1,), jnp.int32),
            pltpu.VMEM((KNN,), jnp.int32),
            pltpu.VMEM((KNN, 128), _F32),
            pltpu.SemaphoreType.DMA,
        ],
    )
    return fn(words2d, aflat)


def _ln_rep(x, g, b):
    mu = jnp.mean(x, axis=-1, keepdims=True)
    var = jnp.mean((x - mu) ** 2, axis=-1, keepdims=True)
    return (x - mu) / jnp.sqrt(var + 1e-5) * g + b


def _k3_body(g_ref, cen_ref, w1_ref, b1_ref, w2_ref, b2_ref, w3_ref, b3_ref,
             pe_ref,
             ln1g_ref, ln1b_ref, wqkv_ref, bqkv_ref, wo_ref, bo_ref,
             ln2g_ref, ln2b_ref, fw1_ref, fb1_ref, fw2_ref, fb2_ref,
             pw_ref, pb_ref, hw1_ref, hb1_ref, hw2_ref, hb2_ref,
             tab_ref):
    Gx = g_ref[0][:, :3]
    C = cen_ref[0]
    crep = jnp.broadcast_to(C.reshape(MP, 1, 3),
                            (MP, KNN, 3)).reshape(MP * KNN, 3)
    local = Gx - crep
    f1 = jnp.maximum(_dot(local, w1_ref[...], _PREC_DIST) + b1_ref[...], 0.0)
    f2 = jnp.maximum(_dot(f1, w2_ref[...], _PREC_DIST) + b2_ref[...], 0.0)
    f3 = _dot(f2, w3_ref[...], _PREC_DIST) + b3_ref[...]
    tok = jnp.max(f3.reshape(MP, KNN, EMB), axis=1)
    tok = tok + pe_ref[0]

    scale = 1.0 / float(np.sqrt(DH))
    for l in range(DEPTH):
        h = _ln_rep(tok, ln1g_ref[l], ln1b_ref[l])
        qkv = _dot(h, wqkv_ref[l], _PREC_DIST) + bqkv_ref[l]
        q, k, v = qkv[:, :EMB], qkv[:, EMB:2 * EMB], qkv[:, 2 * EMB:]
        outs = []
        for hh in range(NHEAD):
            sl = slice(hh * DH, (hh + 1) * DH)
            qh, kh, vh = q[:, sl], k[:, sl], v[:, sl]
            att = lax.dot_general(qh, kh, (((1,), (1,)), ((), ())),
                                  preferred_element_type=_F32,
                                  precision=_PREC_DIST) * scale
            mx = jnp.max(att, axis=-1, keepdims=True)
            e = jnp.exp(att - mx)
            att = e / jnp.sum(e, axis=-1, keepdims=True)
            outs.append(_dot(att, vh, _PREC_DIST))
        o = jnp.concatenate(outs, axis=1)
        tok = tok + _dot(o, wo_ref[l], _PREC_DIST) + bo_ref[l]
        h2 = _ln_rep(tok, ln2g_ref[l], ln2b_ref[l])
        tok = tok + (_dot(jnp.maximum(_dot(h2, fw1_ref[l], _PREC_DIST)
                                      + fb1_ref[l], 0.0),
                          fw2_ref[l], _PREC_DIST) + fb2_ref[l])

    feats = _dot(tok, pw_ref[...], _PREC_DIST) + pb_ref[...]
    t3 = jnp.maximum(_dot(feats, hw1_ref[...], _PREC_DIST) + hb1_ref[...], 0.0)
    tab_ref[0] = _dot(t3, hw2_ref[...], _PREC_DIST) + hb2_ref[...]


def _run_k3(G, centers, args):
    def full(a):
        nd = a.ndim
        return pl.BlockSpec(a.shape, lambda b, _n=nd: (0,) * _n)

    weights = args
    return pl.pallas_call(
        _k3_body,
        grid=(B,),
        in_specs=[pl.BlockSpec((1, MP * KNN, 128), lambda b: (b, 0, 0)),
                  pl.BlockSpec((1, MP, 3), lambda b: (b, 0, 0))] +
                 [full(w) for w in weights],
        out_specs=[pl.BlockSpec((1, MP, NC), lambda b: (b, 0, 0))],
        out_shape=[jax.ShapeDtypeStruct((B, MP, NC), _F32)],
    )(G, centers, *weights)[0]


K4_BLK = 4096


def _k4_body(xyz_ref, cenT_ref, tab_ref, out_ref):
    X = xyz_ref[0]
    Ct = cenT_ref[0]
    xx = jnp.sum(X * X, axis=1, keepdims=True)
    cc = jnp.sum(Ct * Ct, axis=0, keepdims=True)
    P = lax.dot_general(X, Ct, (((1,), (0,)), ((), ())),
                        preferred_element_type=_F32, precision=_PREC_DIST)
    sq = (xx + cc) - 2.0 * P
    d = jnp.maximum(sq, 0.0)
    mn = jnp.min(d, axis=1, keepdims=True)
    li = lax.broadcasted_iota(jnp.int32, (K4_BLK, MP), 1)
    sel = jnp.where(d == mn, li, MP)
    am = jnp.min(sel, axis=1, keepdims=True)
    oh = jnp.where(li == am, 1.0, 0.0)
    out_ref[0] = lax.dot_general(oh, tab_ref[0], (((1,), (0,)), ((), ())),
                                 preferred_element_type=_F32,
                                 precision=lax.Precision.HIGHEST)


def _run_k4(xyz, cenT, tab):
    return pl.pallas_call(
        _k4_body,
        grid=(B, N // K4_BLK),
        in_specs=[
            pl.BlockSpec((1, K4_BLK, 3), lambda b, n: (b, n, 0)),
            pl.BlockSpec((1, 3, MP), lambda b, n: (b, 0, 0)),
            pl.BlockSpec((1, MP, NC), lambda b, n: (b, 0, 0)),
        ],
        out_specs=[pl.BlockSpec((1, K4_BLK, NC), lambda b, n: (b, n, 0))],
        out_shape=[jax.ShapeDtypeStruct((B, N, NC), _F32)],
    )(xyz, cenT, tab)[0]


def kernel(xyz, patch_W1, patch_b1, patch_W2, patch_b2, patch_W3, patch_b3,
           pe, ln1_g, ln1_b, Wqkv, bqkv, Wo, bo, ln2_g, ln2_b,
           ffn_W1, ffn_b1, ffn_W2, ffn_b2, proj_W, proj_b,
           head_W1, head_b1, head_W2, head_b2):
    idx_c = jnp.linspace(0.0, N - 1, MP).astype(jnp.int32)
    centers = jnp.take(xyz, idx_c, axis=1)
    xyzT = jnp.swapaxes(xyz, 1, 2)
    cenT = jnp.swapaxes(centers, 1, 2)
    xyzpad = jnp.pad(xyz, ((0, 0), (0, 0), (0, 125))).reshape(B * N, 128)

    words = _run_k1(xyzT, centers)
    G = _sc_select_gather(words.reshape(B * MP, N // 16), xyzpad)
    tab = _run_k3(G.reshape(B, MP * KNN, 128), centers,
                  (patch_W1, patch_b1, patch_W2, patch_b2, patch_W3, patch_b3,
                   pe, ln1_g, ln1_b, Wqkv, bqkv, Wo, bo, ln2_g, ln2_b,
                   ffn_W1, ffn_b1, ffn_W2, ffn_b2,
                   proj_W, proj_b, head_W1, head_b1, head_W2, head_b2))
    return _run_k4(xyz, cenT, tab)


# speedup vs baseline: 7.1588x; 1.1288x over previous
import functools

import jax
import jax.numpy as jnp
import numpy as np
from jax import lax
from jax.experimental import pallas as pl
from jax.experimental.pallas import tpu as pltpu
from jax.experimental.pallas import tpu_sc as plsc

B, N = 4, 32768
MP, KNN = 64, 128
EMB, DFF, NHEAD, DEPTH, NC = 256, 512, 8, 6, 10
DH = EMB // NHEAD
NW = 32
ROWS_PER_W = (B * MP) // NW

_PREC = lax.Precision.HIGHEST
_PREC_DIST = lax.Precision.DEFAULT
_F32 = jnp.float32


def _dot(a, b, prec=_PREC):
    return lax.dot_general(a, b, (((a.ndim - 1,), (0,)), ((), ())),
                           preferred_element_type=_F32, precision=prec)


MP_BLK = 16


def _k1_body(xyzT_ref, cen_ref, w_ref):
    Xt = xyzT_ref[0]
    C = cen_ref[0]
    xx = jnp.sum(Xt * Xt, axis=0, keepdims=True)
    cc = jnp.sum(C * C, axis=1, keepdims=True)
    P = lax.dot_general(C, Xt, (((1,), (0,)), ((), ())),
                        preferred_element_type=_F32, precision=_PREC_DIST)
    sq = (xx + cc) - 2.0 * P
    dbits = lax.bitcast_convert_type(jnp.maximum(sq, 0.0), jnp.int32)

    def vstep(i, V):
        Vc = V | lax.shift_left(1, 30 - i)
        cnt = jnp.sum(jnp.where(dbits < Vc, 1.0, 0.0), axis=1, keepdims=True)
        return jnp.where(cnt <= 127.0, Vc, V)

    V = lax.fori_loop(0, 31, vstep, jnp.zeros((MP_BLK, 1), jnp.int32))
    cnt_less = jnp.sum(jnp.where(dbits < V, 1.0, 0.0), axis=1, keepdims=True)
    need_eq = 128.0 - cnt_less
    ii = lax.broadcasted_iota(jnp.int32, (1, N), 1)

    def istep(i, I):
        Ic = I | lax.shift_left(1, 14 - i)
        cnt = jnp.sum(jnp.where((dbits == V) & (ii < Ic), 1.0, 0.0),
                      axis=1, keepdims=True)
        return jnp.where(cnt <= need_eq - 1.0, Ic, I)

    I = lax.fori_loop(0, 15, istep, jnp.zeros((MP_BLK, 1), jnp.int32))

    mask = (dbits < V) | ((dbits == V) & (ii <= I))
    words = jnp.zeros((MP_BLK, N // 16), _F32)
    for k in range(16):
        words = words + jnp.where(
            mask[:, k * (N // 16):(k + 1) * (N // 16)], float(1 << k), 0.0)
    w_ref[0] = words.astype(jnp.int32)


def _run_k1(xyzT, centers):
    return pl.pallas_call(
        _k1_body,
        grid=(B, MP // MP_BLK),
        in_specs=[
            pl.BlockSpec((1, 3, N), lambda b, m: (b, 0, 0)),
            pl.BlockSpec((1, MP_BLK, 3), lambda b, m: (b, m, 0)),
        ],
        out_specs=[
            pl.BlockSpec((1, MP_BLK, N // 16), lambda b, m: (b, m, 0)),
        ],
        out_shape=[
            jax.ShapeDtypeStruct((B, MP, N // 16), jnp.int32),
        ],
    )(xyzT, centers)[0]


_GDN = lax.GatherDimensionNumbers(offset_dims=(), collapsed_slice_dims=(0,),
                                  start_index_map=(0,))


def _gather16(vec, idx):
    return lax.gather(vec, idx.reshape(16, 1), _GDN, slice_sizes=(1,),
                      mode=lax.GatherScatterMode.PROMISE_IN_BOUNDS)


NWORDS = N // 16


def _sc_body(w_hbm, a_hbm, out_hbm, mrow, idxpad, idx128, rows, sem):
    cid = lax.axis_index("c")
    sid = lax.axis_index("s")
    wid = sid * 2 + cid
    lane16 = lax.iota(jnp.int32, 16)

    def row_body(j, carry):
        r = wid * ROWS_PER_W + j
        bb = r // MP
        pltpu.sync_copy(w_hbm.at[r], mrow)
        boff = bb * N

        def step(s, acc):
            wv = mrow[pl.ds(s * 16, 16)]
            t = wv
            for k in (1, 2, 4, 8):
                t = t | _gather16(t, (lane16 + k) & 15)

            def slow(a0):
                a = a0
                for tt in range(16):
                    w0 = wv[tt]
                    ebase = (s * 16 + tt) + boff
                    for bit in range(16):
                        idxpad[a] = ebase + bit * (N // 16)
                        a = a + (lax.shift_right_logical(w0, bit) & 1)
                return a

            return lax.cond(t[0] != 0, slow, lambda a0: a0, acc)

        lax.fori_loop(0, NWORDS // 16, step, 0)
        for c in range(8):
            v = jnp.zeros((16,), jnp.int32)
            for tt in range(16):
                v = jnp.where(lane16 == tt, idxpad[c * 16 + tt], v)
            idx128[pl.ds(c * 16, 16)] = v
        pltpu.async_copy(a_hbm.at[idx128], rows, sem).wait()
        dst = out_hbm.at[pl.ds(pl.multiple_of(r * KNN, KNN), KNN)]
        pltpu.sync_copy(rows, dst)
        return carry

    lax.fori_loop(0, ROWS_PER_W, row_body, 0)


def _sc_select_gather(words2d, aflat):
    mesh = plsc.VectorSubcoreMesh(core_axis_name="c", subcore_axis_name="s")
    fn = pl.kernel(
        _sc_body,
        mesh=mesh,
        out_type=jax.ShapeDtypeStruct((B * MP * KNN, 128), _F32),
        scratch_types=[
            pltpu.VMEM((NWORDS,), jnp.int32),
            pltpu.SMEM((KNN + 1,), jnp.int32),
            pltpu.VMEM((KNN,), jnp.int32),
            pltpu.VMEM((KNN, 128), _F32),
            pltpu.SemaphoreType.DMA,
        ],
    )
    return fn(words2d, aflat)


def _ln_rep(x, g, b):
    mu = jnp.mean(x, axis=-1, keepdims=True)
    var = jnp.mean((x - mu) ** 2, axis=-1, keepdims=True)
    return (x - mu) / jnp.sqrt(var + 1e-5) * g + b


def _k3_body(g_ref, cen_ref, w1_ref, b1_ref, w2_ref, b2_ref, w3_ref, b3_ref,
             pe_ref,
             ln1g_ref, ln1b_ref, wqkv_ref, bqkv_ref, wo_ref, bo_ref,
             ln2g_ref, ln2b_ref, fw1_ref, fb1_ref, fw2_ref, fb2_ref,
             pw_ref, pb_ref, hw1_ref, hb1_ref, hw2_ref, hb2_ref,
             tab_ref):
    Gx = g_ref[0][:, :3]
    C = cen_ref[0]
    crep = jnp.broadcast_to(C.reshape(MP, 1, 3),
                            (MP, KNN, 3)).reshape(MP * KNN, 3)
    local = Gx - crep
    f1 = jnp.maximum(_dot(local, w1_ref[...], _PREC_DIST) + b1_ref[...], 0.0)
    f2 = jnp.maximum(_dot(f1, w2_ref[...], _PREC_DIST) + b2_ref[...], 0.0)
    f3 = _dot(f2, w3_ref[...], _PREC_DIST) + b3_ref[...]
    tok = jnp.max(f3.reshape(MP, KNN, EMB), axis=1)
    tok = tok + pe_ref[0]

    scale = 1.0 / float(np.sqrt(DH))
    for l in range(DEPTH):
        h = _ln_rep(tok, ln1g_ref[l], ln1b_ref[l])
        qkv = _dot(h, wqkv_ref[l], _PREC_DIST) + bqkv_ref[l]
        q, k, v = qkv[:, :EMB], qkv[:, EMB:2 * EMB], qkv[:, 2 * EMB:]
        outs = []
        for hh in range(NHEAD):
            sl = slice(hh * DH, (hh + 1) * DH)
            qh, kh, vh = q[:, sl], k[:, sl], v[:, sl]
            att = lax.dot_general(qh, kh, (((1,), (1,)), ((), ())),
                                  preferred_element_type=_F32,
                                  precision=_PREC_DIST) * scale
            mx = jnp.max(att, axis=-1, keepdims=True)
            e = jnp.exp(att - mx)
            att = e / jnp.sum(e, axis=-1, keepdims=True)
            outs.append(_dot(att, vh, _PREC_DIST))
        o = jnp.concatenate(outs, axis=1)
        tok = tok + _dot(o, wo_ref[l], _PREC_DIST) + bo_ref[l]
        h2 = _ln_rep(tok, ln2g_ref[l], ln2b_ref[l])
        tok = tok + (_dot(jnp.maximum(_dot(h2, fw1_ref[l], _PREC_DIST)
                                      + fb1_ref[l], 0.0),
                          fw2_ref[l], _PREC_DIST) + fb2_ref[l])

    feats = _dot(tok, pw_ref[...], _PREC_DIST) + pb_ref[...]
    t3 = jnp.maximum(_dot(feats, hw1_ref[...], _PREC_DIST) + hb1_ref[...], 0.0)
    tab_ref[0] = _dot(t3, hw2_ref[...], _PREC_DIST) + hb2_ref[...]


def _run_k3(G, centers, args):
    def full(a):
        nd = a.ndim
        return pl.BlockSpec(a.shape, lambda b, _n=nd: (0,) * _n)

    weights = args
    return pl.pallas_call(
        _k3_body,
        grid=(B,),
        in_specs=[pl.BlockSpec((1, MP * KNN, 128), lambda b: (b, 0, 0)),
                  pl.BlockSpec((1, MP, 3), lambda b: (b, 0, 0))] +
                 [full(w) for w in weights],
        out_specs=[pl.BlockSpec((1, MP, NC), lambda b: (b, 0, 0))],
        out_shape=[jax.ShapeDtypeStruct((B, MP, NC), _F32)],
    )(G, centers, *weights)[0]


K4_BLK = 4096


def _k4_body(xyz_ref, cenT_ref, tab_ref, out_ref):
    X = xyz_ref[0]
    Ct = cenT_ref[0]
    xx = jnp.sum(X * X, axis=1, keepdims=True)
    cc = jnp.sum(Ct * Ct, axis=0, keepdims=True)
    P = lax.dot_general(X, Ct, (((1,), (0,)), ((), ())),
                        preferred_element_type=_F32, precision=_PREC_DIST)
    sq = (xx + cc) - 2.0 * P
    d = jnp.maximum(sq, 0.0)
    mn = jnp.min(d, axis=1, keepdims=True)
    li = lax.broadcasted_iota(jnp.int32, (K4_BLK, MP), 1)
    sel = jnp.where(d == mn, li, MP)
    am = jnp.min(sel, axis=1, keepdims=True)
    oh = jnp.where(li == am, 1.0, 0.0)
    out_ref[0] = lax.dot_general(oh, tab_ref[0], (((1,), (0,)), ((), ())),
                                 preferred_element_type=_F32,
                                 precision=lax.Precision.HIGHEST)


def _run_k4(xyz, cenT, tab):
    return pl.pallas_call(
        _k4_body,
        grid=(B, N // K4_BLK),
        in_specs=[
            pl.BlockSpec((1, K4_BLK, 3), lambda b, n: (b, n, 0)),
            pl.BlockSpec((1, 3, MP), lambda b, n: (b, 0, 0)),
            pl.BlockSpec((1, MP, NC), lambda b, n: (b, 0, 0)),
        ],
        out_specs=[pl.BlockSpec((1, K4_BLK, NC), lambda b, n: (b, n, 0))],
        out_shape=[jax.ShapeDtypeStruct((B, N, NC), _F32)],
    )(xyz, cenT, tab)[0]


def kernel(xyz, patch_W1, patch_b1, patch_W2, patch_b2, patch_W3, patch_b3,
           pe, ln1_g, ln1_b, Wqkv, bqkv, Wo, bo, ln2_g, ln2_b,
           ffn_W1, ffn_b1, ffn_W2, ffn_b2, proj_W, proj_b,
           head_W1, head_b1, head_W2, head_b2):
    idx_c = jnp.linspace(0.0, N - 1, MP).astype(jnp.int32)
    centers = jnp.take(xyz, idx_c, axis=1)
    xyzT = jnp.swapaxes(xyz, 1, 2)
    cenT = jnp.swapaxes(centers, 1, 2)
    xyzpad = jnp.pad(xyz, ((0, 0), (0, 0), (0, 125))).reshape(B * N, 128)

    words = _run_k1(xyzT, centers)
    G = _sc_select_gather(words.reshape(B * MP, N // 16), xyzpad)
    tab = _run_k3(G.reshape(B, MP * KNN, 128), centers,
                  (patch_W1, patch_b1, patch_W2, patch_b2, patch_W3, patch_b3,
                   pe, ln1_g, ln1_b, Wqkv, bqkv, Wo, bo, ln2_g, ln2_b,
                   ffn_W1, ffn_b1, ffn_W2, ffn_b2,
                   proj_W, proj_b, head_W1, head_b1, head_W2, head_b2))
    return _run_k4(xyz, cenT, tab)


# speedup vs baseline: 7.5990x; 1.0615x over previous
import functools

import jax
import jax.numpy as jnp
import numpy as np
from jax import lax
from jax.experimental import pallas as pl
from jax.experimental.pallas import tpu as pltpu
from jax.experimental.pallas import tpu_sc as plsc

B, N = 4, 32768
MP, KNN = 64, 128
EMB, DFF, NHEAD, DEPTH, NC = 256, 512, 8, 6, 10
DH = EMB // NHEAD
NW = 32
ROWS_PER_W = (B * MP) // NW

_PREC = lax.Precision.HIGHEST
_PREC_DIST = lax.Precision.DEFAULT
_F32 = jnp.float32


def _dot(a, b, prec=_PREC):
    return lax.dot_general(a, b, (((a.ndim - 1,), (0,)), ((), ())),
                           preferred_element_type=_F32, precision=prec)


MP_BLK = 16


def _k1_body(xyzT_ref, cen_ref, w_ref):
    Xt = xyzT_ref[0]
    C = cen_ref[0]
    xx = jnp.sum(Xt * Xt, axis=0, keepdims=True)
    cc = jnp.sum(C * C, axis=1, keepdims=True)
    P = lax.dot_general(C, Xt, (((1,), (0,)), ((), ())),
                        preferred_element_type=_F32, precision=_PREC_DIST)
    sq = (xx + cc) - 2.0 * P
    dbits = lax.bitcast_convert_type(jnp.maximum(sq, 0.0), jnp.int32)

    def vstep(i, V):
        Vc = V | lax.shift_left(1, 30 - i)
        cnt = jnp.sum(jnp.where(dbits < Vc, 1.0, 0.0), axis=1, keepdims=True)
        return jnp.where(cnt <= 127.0, Vc, V)

    V = lax.fori_loop(0, 31, vstep, jnp.zeros((MP_BLK, 1), jnp.int32))
    cnt_less = jnp.sum(jnp.where(dbits < V, 1.0, 0.0), axis=1, keepdims=True)
    need_eq = 128.0 - cnt_less
    ii = lax.broadcasted_iota(jnp.int32, (1, N), 1)

    def istep(i, I):
        Ic = I | lax.shift_left(1, 14 - i)
        cnt = jnp.sum(jnp.where((dbits == V) & (ii < Ic), 1.0, 0.0),
                      axis=1, keepdims=True)
        return jnp.where(cnt <= need_eq - 1.0, Ic, I)

    I = lax.fori_loop(0, 15, istep, jnp.zeros((MP_BLK, 1), jnp.int32))

    mask = (dbits < V) | ((dbits == V) & (ii <= I))
    words = jnp.zeros((MP_BLK, N // 16), _F32)
    for k in range(16):
        words = words + jnp.where(
            mask[:, k * (N // 16):(k + 1) * (N // 16)], float(1 << k), 0.0)
    w_ref[0] = words.astype(jnp.int32)


def _run_k1(xyzT, centers):
    return pl.pallas_call(
        _k1_body,
        grid=(B, MP // MP_BLK),
        in_specs=[
            pl.BlockSpec((1, 3, N), lambda b, m: (b, 0, 0)),
            pl.BlockSpec((1, MP_BLK, 3), lambda b, m: (b, m, 0)),
        ],
        out_specs=[
            pl.BlockSpec((1, MP_BLK, N // 16), lambda b, m: (b, m, 0)),
        ],
        out_shape=[
            jax.ShapeDtypeStruct((B, MP, N // 16), jnp.int32),
        ],
    )(xyzT, centers)[0]


_GDN = lax.GatherDimensionNumbers(offset_dims=(), collapsed_slice_dims=(0,),
                                  start_index_map=(0,))


def _gather16(vec, idx):
    return lax.gather(vec, idx.reshape(16, 1), _GDN, slice_sizes=(1,),
                      mode=lax.GatherScatterMode.PROMISE_IN_BOUNDS)


NWORDS = N // 16


def _sc_body(w_hbm, a_hbm, out_hbm, mrow, idxpad, idx128, rows, sem):
    cid = lax.axis_index("c")
    sid = lax.axis_index("s")
    wid = sid * 2 + cid
    lane16 = lax.iota(jnp.int32, 16)

    def row_body(j, carry):
        r = wid * ROWS_PER_W + j
        bb = r // MP
        pltpu.sync_copy(w_hbm.at[r], mrow)
        boff = bb * N

        def step(s, acc):
            wv = mrow[pl.ds(s * 16, 16)]
            t = wv
            for k in (1, 2, 4, 8):
                t = t | _gather16(t, (lane16 + k) & 15)

            def slow(a0):
                a = a0
                for tt in range(16):
                    w0 = wv[tt]
                    ebase = (s * 16 + tt) + boff
                    for bit in range(16):
                        idxpad[a] = ebase + bit * (N // 16)
                        a = a + (lax.shift_right_logical(w0, bit) & 1)
                return a

            return lax.cond(t[0] != 0, slow, lambda a0: a0, acc)

        lax.fori_loop(0, NWORDS // 16, step, 0)
        for c in range(8):
            v = jnp.zeros((16,), jnp.int32)
            for tt in range(16):
                v = jnp.where(lane16 == tt, idxpad[c * 16 + tt], v)
            idx128[pl.ds(c * 16, 16)] = v
        pltpu.async_copy(a_hbm.at[idx128], rows, sem).wait()
        dst = out_hbm.at[pl.ds(pl.multiple_of(r * KNN, KNN), KNN)]
        pltpu.sync_copy(rows, dst)
        return carry

    lax.fori_loop(0, ROWS_PER_W, row_body, 0)


def _sc_select_gather(words2d, aflat):
    mesh = plsc.VectorSubcoreMesh(core_axis_name="c", subcore_axis_name="s")
    fn = pl.kernel(
        _sc_body,
        mesh=mesh,
        out_type=jax.ShapeDtypeStruct((B * MP * KNN, 128), _F32),
        scratch_types=[
            pltpu.VMEM((NWORDS,), jnp.int32),
            pltpu.SMEM((KNN + 1,), jnp.int32),
            pltpu.VMEM((KNN,), jnp.int32),
            pltpu.VMEM((KNN, 128), _F32),
            pltpu.SemaphoreType.DMA,
        ],
    )
    return fn(words2d, aflat)


def _ln_rep(x, g, b):
    mu = jnp.mean(x, axis=-1, keepdims=True)
    var = jnp.mean((x - mu) ** 2, axis=-1, keepdims=True)
    return (x - mu) / jnp.sqrt(var + 1e-5) * g + b


def _k3_body(g_ref, cen_ref, w1_ref, b1_ref, w2_ref, b2_ref, w3_ref, b3_ref,
             pe_ref,
             ln1g_ref, ln1b_ref, wqkv_ref, bqkv_ref, wo_ref, bo_ref,
             ln2g_ref, ln2b_ref, fw1_ref, fb1_ref, fw2_ref, fb2_ref,
             pw_ref, pb_ref, hw1_ref, hb1_ref, hw2_ref, hb2_ref,
             tab_ref):
    Gx = g_ref[0][:, :3]
    C = cen_ref[0]
    crep = jnp.broadcast_to(C.reshape(MP, 1, 3),
                            (MP, KNN, 3)).reshape(MP * KNN, 3)
    local = Gx - crep
    f1 = jnp.maximum(_dot(local, w1_ref[...], _PREC_DIST) + b1_ref[...], 0.0)
    f2 = jnp.maximum(_dot(f1, w2_ref[...], _PREC_DIST) + b2_ref[...], 0.0)
    f3 = _dot(f2, w3_ref[...], _PREC_DIST) + b3_ref[...]
    tok = jnp.max(f3.reshape(MP, KNN, EMB), axis=1)
    tok = tok + pe_ref[0]

    scale = 1.0 / float(np.sqrt(DH))
    for l in range(DEPTH):
        h = _ln_rep(tok, ln1g_ref[l], ln1b_ref[l])
        qkv = _dot(h, wqkv_ref[l], _PREC_DIST) + bqkv_ref[l]
        q, k, v = qkv[:, :EMB], qkv[:, EMB:2 * EMB], qkv[:, 2 * EMB:]
        outs = []
        for hh in range(NHEAD):
            sl = slice(hh * DH, (hh + 1) * DH)
            qh, kh, vh = q[:, sl], k[:, sl], v[:, sl]
            att = lax.dot_general(qh, kh, (((1,), (1,)), ((), ())),
                                  preferred_element_type=_F32,
                                  precision=_PREC_DIST) * scale
            mx = jnp.max(att, axis=-1, keepdims=True)
            e = jnp.exp(att - mx)
            att = e / jnp.sum(e, axis=-1, keepdims=True)
            outs.append(_dot(att, vh, _PREC_DIST))
        o = jnp.concatenate(outs, axis=1)
        tok = tok + _dot(o, wo_ref[l], _PREC_DIST) + bo_ref[l]
        h2 = _ln_rep(tok, ln2g_ref[l], ln2b_ref[l])
        tok = tok + (_dot(jnp.maximum(_dot(h2, fw1_ref[l], _PREC_DIST)
                                      + fb1_ref[l], 0.0),
                          fw2_ref[l], _PREC_DIST) + fb2_ref[l])

    feats = _dot(tok, pw_ref[...], _PREC_DIST) + pb_ref[...]
    t3 = jnp.maximum(_dot(feats, hw1_ref[...], _PREC_DIST) + hb1_ref[...], 0.0)
    tab_ref[0] = _dot(t3, hw2_ref[...], _PREC_DIST) + hb2_ref[...]


def _run_k3(G, centers, args):
    def full(a):
        nd = a.ndim
        return pl.BlockSpec(a.shape, lambda b, _n=nd: (0,) * _n)

    weights = args
    return pl.pallas_call(
        _k3_body,
        grid=(B,),
        in_specs=[pl.BlockSpec((1, MP * KNN, 128), lambda b: (b, 0, 0)),
                  pl.BlockSpec((1, MP, 3), lambda b: (b, 0, 0))] +
                 [full(w) for w in weights],
        out_specs=[pl.BlockSpec((1, MP, NC), lambda b: (b, 0, 0))],
        out_shape=[jax.ShapeDtypeStruct((B, MP, NC), _F32)],
    )(G, centers, *weights)[0]


K4_BLK = 4096


def _k4a_body(xyz_ref, cenT_ref, am_ref):
    X = xyz_ref[0]
    Ct = cenT_ref[0]
    xx = jnp.sum(X * X, axis=1, keepdims=True)
    cc = jnp.sum(Ct * Ct, axis=0, keepdims=True)
    P = lax.dot_general(X, Ct, (((1,), (0,)), ((), ())),
                        preferred_element_type=_F32, precision=_PREC_DIST)
    sq = (xx + cc) - 2.0 * P
    d = jnp.maximum(sq, 0.0)
    mn = jnp.min(d, axis=1, keepdims=True)
    li = lax.broadcasted_iota(jnp.int32, (K4_BLK, MP), 1)
    sel = jnp.where(d == mn, li, MP)
    am_ref[0] = jnp.min(sel, axis=1, keepdims=True)


def _run_k4a(xyz, cenT):
    return pl.pallas_call(
        _k4a_body,
        grid=(B, N // K4_BLK),
        in_specs=[
            pl.BlockSpec((1, K4_BLK, 3), lambda b, n: (b, n, 0)),
            pl.BlockSpec((1, 3, MP), lambda b, n: (b, 0, 0)),
        ],
        out_specs=[pl.BlockSpec((1, K4_BLK, 1), lambda b, n: (b, n, 0))],
        out_shape=[jax.ShapeDtypeStruct((B, N, 1), jnp.int32)],
    )(xyz, cenT)[0]


def _k4b_body(am_ref, tab_ref, out_ref):
    am = am_ref[0]
    li = lax.broadcasted_iota(jnp.int32, (K4_BLK, MP), 1)
    oh = jnp.where(li == am, 1.0, 0.0)
    out_ref[0] = lax.dot_general(oh, tab_ref[0], (((1,), (0,)), ((), ())),
                                 preferred_element_type=_F32,
                                 precision=lax.Precision.HIGHEST)


def _run_k4b(am, tab):
    return pl.pallas_call(
        _k4b_body,
        grid=(B, N // K4_BLK),
        in_specs=[
            pl.BlockSpec((1, K4_BLK, 1), lambda b, n: (b, n, 0)),
            pl.BlockSpec((1, MP, NC), lambda b, n: (b, 0, 0)),
        ],
        out_specs=[pl.BlockSpec((1, K4_BLK, NC), lambda b, n: (b, n, 0))],
        out_shape=[jax.ShapeDtypeStruct((B, N, NC), _F32)],
    )(am, tab)[0]


def kernel(xyz, patch_W1, patch_b1, patch_W2, patch_b2, patch_W3, patch_b3,
           pe, ln1_g, ln1_b, Wqkv, bqkv, Wo, bo, ln2_g, ln2_b,
           ffn_W1, ffn_b1, ffn_W2, ffn_b2, proj_W, proj_b,
           head_W1, head_b1, head_W2, head_b2):
    idx_c = jnp.linspace(0.0, N - 1, MP).astype(jnp.int32)
    centers = jnp.take(xyz, idx_c, axis=1)
    xyzT = jnp.swapaxes(xyz, 1, 2)
    cenT = jnp.swapaxes(centers, 1, 2)
    xyzpad = jnp.pad(xyz, ((0, 0), (0, 0), (0, 125))).reshape(B * N, 128)

    words = _run_k1(xyzT, centers)
    G = _sc_select_gather(words.reshape(B * MP, N // 16), xyzpad)
    am = _run_k4a(xyz, cenT)
    tab = _run_k3(G.reshape(B, MP * KNN, 128), centers,
                  (patch_W1, patch_b1, patch_W2, patch_b2, patch_W3, patch_b3,
                   pe, ln1_g, ln1_b, Wqkv, bqkv, Wo, bo, ln2_g, ln2_b,
                   ffn_W1, ffn_b1, ffn_W2, ffn_b2,
                   proj_W, proj_b, head_W1, head_b1, head_W2, head_b2))
    return _run_k4b(am, tab)
